# baseline (speedup 1.0000x reference)
import functools

import jax
import jax.numpy as jnp
from jax import lax
from jax.experimental import pallas as pl
from jax.experimental.pallas import tpu as pltpu
from jax.experimental.pallas import tpu_sc as plsc

_NUM_CLASS = 1024
_NUM_FEAT = 8192
_DIM = 64
_TOP_K = 3
_COMMIT = 0.25
_T = 257
_B = 16
_KTOT = _NUM_CLASS + _NUM_FEAT
_FMAX = float(jnp.finfo(jnp.float32).max)
_TBLK = 8
_ROWS = _TBLK * _B
_NSTEP = (_T + _TBLK - 1) // _TBLK


def _top3(d, width):
    iot = lax.broadcasted_iota(jnp.int32, d.shape, 1)
    cur = d
    vals, idxs = [], []
    onehot = jnp.zeros(d.shape, jnp.float32)
    for _ in range(_TOP_K):
        ii = jnp.argmin(cur, axis=1, keepdims=True).astype(jnp.int32)
        oh = iot == ii
        m = jnp.min(cur, axis=1, keepdims=True)
        onehot = onehot + oh.astype(jnp.float32)
        vals.append(m)
        idxs.append(ii)
        cur = jnp.where(oh, _FMAX, cur)
    return vals, idxs, onehot


def _row_norms_lanes(cb):
    sq = cb * cb
    ones = jnp.ones((1, _DIM), jnp.float32)
    return lax.dot_general(ones, sq, (((1,), (1,)), ((), ())),
                           preferred_element_type=jnp.float32)


def _pack3(idxs, rows):
    lane = lax.broadcasted_iota(jnp.int32, (rows, _TOP_K), 1)
    return jnp.where(lane == 0, idxs[0],
                     jnp.where(lane == 1, idxs[1], idxs[2]))


def _tc_body(feat_ref, ccb_ref, fcb_ref,
             dist_ref, idx_ref, loss_ref, cperp_ref, fperp_ref,
             c2f, cnt_c, cnt_f, csum, fsum):
    i = pl.program_id(0)

    @pl.when(i == 0)
    def _init():
        c2f[...] = _row_norms_lanes(fcb_ref[...])
        cnt_f[...] = jnp.zeros((_ROWS, _NUM_FEAT), jnp.float32)
        fsum[...] = jnp.zeros((1, 1), jnp.float32)

    x = feat_ref[...].reshape(_ROWS, _DIM)
    m = lax.dot_general(x, fcb_ref[...], (((1,), (1,)), ((), ())),
                        preferred_element_type=jnp.float32)
    x2 = jnp.sum(x * x, axis=1, keepdims=True)
    d = (x2 + c2f[...]) - 2.0 * m

    vals, idxs, onehot = _top3(d, _NUM_FEAT)

    @pl.when((i > 0) & (i < _NSTEP - 1))
    def _acc_interior():
        cnt_f[...] += onehot
        fsum[...] += jnp.sum(vals[0] + vals[1] + vals[2])

    @pl.when((i == 0) | (i == _NSTEP - 1))
    def _acc_boundary():
        r = lax.broadcasted_iota(jnp.int32, (_ROWS, 1), 0)
        g = _TBLK * i + (r >> 4)
        valid = (g >= 1) & (g <= _T - 1)
        cnt_f[...] += onehot * valid.astype(jnp.float32)
        fsum[...] += jnp.sum(
            jnp.where(valid, vals[0] + vals[1] + vals[2], 0.0))

    dist_ref[:, :, :_NUM_CLASS] = jnp.full((_TBLK, _B, _NUM_CLASS), _FMAX,
                                           jnp.float32)
    dist_ref[:, :, _NUM_CLASS:] = d.reshape(_TBLK, _B, _NUM_FEAT)
    idx_ref[...] = (_pack3(idxs, _ROWS) + _NUM_CLASS).reshape(
        _TBLK, _B, _TOP_K)

    @pl.when(i == 0)
    def _class_part():
        xc = feat_ref[0]
        mc = lax.dot_general(xc, ccb_ref[...], (((1,), (1,)), ((), ())),
                             preferred_element_type=jnp.float32)
        xc2 = jnp.sum(xc * xc, axis=1, keepdims=True)
        c2c = _row_norms_lanes(ccb_ref[...])
        dc = (xc2 + c2c) - 2.0 * mc
        vals_c, idxs_c, onehot_c = _top3(dc, _NUM_CLASS)
        dist_ref[0, :, :_NUM_CLASS] = dc
        dist_ref[0, :, _NUM_CLASS:] = jnp.full((_B, _NUM_FEAT), _FMAX,
                                               jnp.float32)
        idx_ref[0] = _pack3(idxs_c, _B)
        cnt_c[...] = onehot_c
        csum[...] = jnp.broadcast_to(
            jnp.sum(vals_c[0] + vals_c[1] + vals_c[2]), (1, 1))

    @pl.when(i == _NSTEP - 1)
    def _finalize():
        n_c = float(_B * _TOP_K * _DIM)
        n_f = float((_T - 1) * _B * _TOP_K * _DIM)
        loss_ref[...] = _COMMIT * (csum[...] / n_c + fsum[...] / n_f)
        avg_c = jnp.sum(cnt_c[...], axis=0, keepdims=True) / float(_B)
        s_c = jnp.sum(avg_c * jnp.log(avg_c + 1e-10))
        cperp_ref[...] = jnp.exp(jnp.broadcast_to(-s_c, (1, 1)))
        avg_f = jnp.sum(cnt_f[...], axis=0, keepdims=True) / float(
            (_T - 1) * _B)
        s_f = jnp.sum(avg_f * jnp.log(avg_f + 1e-10))
        fperp_ref[...] = jnp.exp(jnp.broadcast_to(-s_f, (1, 1)))


def _tc_call(features, class_codebook, feature_codebook):
    return pl.pallas_call(
        _tc_body,
        grid=(_NSTEP,),
        in_specs=[
            pl.BlockSpec((_TBLK, _B, _DIM), lambda i: (i, 0, 0)),
            pl.BlockSpec((_NUM_CLASS, _DIM), lambda i: (0, 0)),
            pl.BlockSpec((_NUM_FEAT, _DIM), lambda i: (0, 0)),
        ],
        out_specs=[
            pl.BlockSpec((_TBLK, _B, _KTOT), lambda i: (i, 0, 0)),
            pl.BlockSpec((_TBLK, _B, _TOP_K), lambda i: (i, 0, 0)),
            pl.BlockSpec((1, 1), lambda i: (0, 0)),
            pl.BlockSpec((1, 1), lambda i: (0, 0)),
            pl.BlockSpec((1, 1), lambda i: (0, 0)),
        ],
        out_shape=[
            jax.ShapeDtypeStruct((_T, _B, _KTOT), jnp.float32),
            jax.ShapeDtypeStruct((_T, _B, _TOP_K), jnp.int32),
            jax.ShapeDtypeStruct((1, 1), jnp.float32),
            jax.ShapeDtypeStruct((1, 1), jnp.float32),
            jax.ShapeDtypeStruct((1, 1), jnp.float32),
        ],
        scratch_shapes=[
            pltpu.VMEM((1, _NUM_FEAT), jnp.float32),
            pltpu.VMEM((_B, _NUM_CLASS), jnp.float32),
            pltpu.VMEM((_ROWS, _NUM_FEAT), jnp.float32),
            pltpu.VMEM((1, 1), jnp.float32),
            pltpu.VMEM((1, 1), jnp.float32),
        ],
        compiler_params=pltpu.CompilerParams(
            dimension_semantics=("arbitrary",)),
    )(features, class_codebook, feature_codebook)


_GATHER_CHUNK = 56


def _sc_gather(table, idx):
    info = plsc.get_sparse_core_info()
    ncores, nsub = info.num_cores, info.num_subcores
    nw = ncores * nsub
    bpw = idx.shape[0] // nw
    dim = table.shape[1]
    mesh = plsc.VectorSubcoreMesh(core_axis_name="c", subcore_axis_name="s")

    @functools.partial(
        pl.kernel, mesh=mesh,
        out_type=jax.ShapeDtypeStruct((idx.shape[0], dim), jnp.float32),
        scratch_types=[
            pltpu.VMEM((bpw,), jnp.int32),
            pltpu.VMEM((bpw, dim), jnp.float32),
            pltpu.SemaphoreType.DMA,
        ],
    )
    def k(table_hbm, idx_hbm, out_hbm, idx_v, rows_v, sem):
        wid = lax.axis_index("s") * ncores + lax.axis_index("c")
        base = wid * bpw
        pltpu.sync_copy(idx_hbm.at[pl.ds(base, bpw)], idx_v)
        for j in range(bpw // _GATHER_CHUNK):
            off = j * _GATHER_CHUNK
            pltpu.async_copy(
                table_hbm.at[idx_v.at[pl.ds(off, _GATHER_CHUNK)]],
                rows_v.at[pl.ds(off, _GATHER_CHUNK)], sem).wait()
        pltpu.sync_copy(rows_v, out_hbm.at[pl.ds(base, bpw)])

    return k(table, idx)


def kernel(features, class_codebook, feature_codebook):
    distances, indices, loss11, cperp11, fperp11 = _tc_call(
        features, class_codebook, feature_codebook)

    combined = jnp.concatenate([class_codebook, feature_codebook], axis=0)
    combined = jnp.pad(combined, ((0, 0), (0, 128 - _DIM)))
    nidx = _T * _B * _TOP_K
    npad = 12544
    flat_idx = jnp.concatenate(
        [indices.reshape(-1), jnp.zeros((npad - nidx,), jnp.int32)])
    rows = _sc_gather(combined, flat_idx)
    quantized = rows[:nidx, :_DIM].reshape(_T, _B, _TOP_K, _DIM)

    return (loss11[0, 0], quantized, cperp11[0, 0], fperp11[0, 0],
            indices, distances)


# speedup vs baseline: 1.0076x; 1.0076x over previous
import functools

import jax
import jax.numpy as jnp
from jax import lax
from jax.experimental import pallas as pl
from jax.experimental.pallas import tpu as pltpu
from jax.experimental.pallas import tpu_sc as plsc

_NUM_CLASS = 1024
_NUM_FEAT = 8192
_DIM = 64
_TOP_K = 3
_COMMIT = 0.25
_T = 257
_B = 16
_KTOT = _NUM_CLASS + _NUM_FEAT
_FMAX = float(jnp.finfo(jnp.float32).max)
_TBLK = 8
_ROWS = _TBLK * _B
_NSTEP = (_T + _TBLK - 1) // _TBLK


def _top3(d, width):
    iot = lax.broadcasted_iota(jnp.int32, d.shape, 1)
    cur = d
    vals, idxs = [], []
    onehot = jnp.zeros(d.shape, jnp.float32)
    for _ in range(_TOP_K):
        m = jnp.min(cur, axis=1, keepdims=True)
        ii = jnp.min(jnp.where(cur == m, iot, width), axis=1, keepdims=True)
        oh = iot == ii
        onehot = onehot + oh.astype(jnp.float32)
        vals.append(m)
        idxs.append(ii)
        cur = jnp.where(oh, _FMAX, cur)
    return vals, idxs, onehot


def _pack3(idxs, rows):
    lane = lax.broadcasted_iota(jnp.int32, (rows, _TOP_K), 1)
    return jnp.where(lane == 0, idxs[0],
                     jnp.where(lane == 1, idxs[1], idxs[2]))


def _tc_body(feat_ref, ccb_ref, fcb_ref,
             dist_ref, idx_ref, loss_ref, cperp_ref, fperp_ref,
             c2f, cnt_c, cnt_f, csum, fsum):
    i = pl.program_id(0)

    @pl.when(i == 0)
    def _init():
        fcb = fcb_ref[...]
        c2f[...] = jnp.sum(fcb * fcb, axis=1)[None, :]
        cnt_f[...] = jnp.zeros((_ROWS, _NUM_FEAT), jnp.float32)
        fsum[...] = jnp.zeros((1, 1), jnp.float32)

    x = feat_ref[...].reshape(_ROWS, _DIM)
    m = lax.dot_general(x, fcb_ref[...], (((1,), (1,)), ((), ())),
                        preferred_element_type=jnp.float32)
    x2 = jnp.sum(x * x, axis=1, keepdims=True)
    d = (x2 + c2f[...]) - 2.0 * m

    vals, idxs, onehot = _top3(d, _NUM_FEAT)

    @pl.when((i > 0) & (i < _NSTEP - 1))
    def _acc_interior():
        cnt_f[...] += onehot
        fsum[...] += jnp.sum(vals[0] + vals[1] + vals[2])

    @pl.when((i == 0) | (i == _NSTEP - 1))
    def _acc_boundary():
        r = lax.broadcasted_iota(jnp.int32, (_ROWS, 1), 0)
        g = _TBLK * i + (r >> 4)
        valid = (g >= 1) & (g <= _T - 1)
        cnt_f[...] += onehot * valid.astype(jnp.float32)
        fsum[...] += jnp.sum(
            jnp.where(valid, vals[0] + vals[1] + vals[2], 0.0))

    dist_ref[:, :, :_NUM_CLASS] = jnp.full((_TBLK, _B, _NUM_CLASS), _FMAX,
                                           jnp.float32)
    dist_ref[:, :, _NUM_CLASS:] = d.reshape(_TBLK, _B, _NUM_FEAT)
    idx_ref[...] = (_pack3(idxs, _ROWS) + _NUM_CLASS).reshape(
        _TBLK, _B, _TOP_K)

    @pl.when(i == 0)
    def _class_part():
        xc = feat_ref[0]
        mc = lax.dot_general(xc, ccb_ref[...], (((1,), (1,)), ((), ())),
                             preferred_element_type=jnp.float32)
        xc2 = jnp.sum(xc * xc, axis=1, keepdims=True)
        ccb = ccb_ref[...]
        c2c = jnp.sum(ccb * ccb, axis=1)
        dc = (xc2 + c2c[None, :]) - 2.0 * mc
        vals_c, idxs_c, onehot_c = _top3(dc, _NUM_CLASS)
        dist_ref[0, :, :_NUM_CLASS] = dc
        dist_ref[0, :, _NUM_CLASS:] = jnp.full((_B, _NUM_FEAT), _FMAX,
                                               jnp.float32)
        idx_ref[0] = _pack3(idxs_c, _B)
        cnt_c[...] = onehot_c
        csum[...] = jnp.broadcast_to(
            jnp.sum(vals_c[0] + vals_c[1] + vals_c[2]), (1, 1))

    @pl.when(i == _NSTEP - 1)
    def _finalize():
        n_c = float(_B * _TOP_K * _DIM)
        n_f = float((_T - 1) * _B * _TOP_K * _DIM)
        loss_ref[...] = _COMMIT * (csum[...] / n_c + fsum[...] / n_f)
        avg_c = jnp.sum(cnt_c[...], axis=0, keepdims=True) / float(_B)
        s_c = jnp.sum(avg_c * jnp.log(avg_c + 1e-10))
        cperp_ref[...] = jnp.exp(jnp.broadcast_to(-s_c, (1, 1)))
        avg_f = jnp.sum(cnt_f[...], axis=0, keepdims=True) / float(
            (_T - 1) * _B)
        s_f = jnp.sum(avg_f * jnp.log(avg_f + 1e-10))
        fperp_ref[...] = jnp.exp(jnp.broadcast_to(-s_f, (1, 1)))


def _tc_call(features, class_codebook, feature_codebook):
    return pl.pallas_call(
        _tc_body,
        grid=(_NSTEP,),
        in_specs=[
            pl.BlockSpec((_TBLK, _B, _DIM), lambda i: (i, 0, 0)),
            pl.BlockSpec((_NUM_CLASS, _DIM), lambda i: (0, 0)),
            pl.BlockSpec((_NUM_FEAT, _DIM), lambda i: (0, 0)),
        ],
        out_specs=[
            pl.BlockSpec((_TBLK, _B, _KTOT), lambda i: (i, 0, 0)),
            pl.BlockSpec((_TBLK, _B, _TOP_K), lambda i: (i, 0, 0)),
            pl.BlockSpec((1, 1), lambda i: (0, 0)),
            pl.BlockSpec((1, 1), lambda i: (0, 0)),
            pl.BlockSpec((1, 1), lambda i: (0, 0)),
        ],
        out_shape=[
            jax.ShapeDtypeStruct((_T, _B, _KTOT), jnp.float32),
            jax.ShapeDtypeStruct((_T, _B, _TOP_K), jnp.int32),
            jax.ShapeDtypeStruct((1, 1), jnp.float32),
            jax.ShapeDtypeStruct((1, 1), jnp.float32),
            jax.ShapeDtypeStruct((1, 1), jnp.float32),
        ],
        scratch_shapes=[
            pltpu.VMEM((1, _NUM_FEAT), jnp.float32),
            pltpu.VMEM((_B, _NUM_CLASS), jnp.float32),
            pltpu.VMEM((_ROWS, _NUM_FEAT), jnp.float32),
            pltpu.VMEM((1, 1), jnp.float32),
            pltpu.VMEM((1, 1), jnp.float32),
        ],
        compiler_params=pltpu.CompilerParams(
            dimension_semantics=("arbitrary",)),
    )(features, class_codebook, feature_codebook)


_GATHER_CHUNK = 56


def _sc_gather(table, idx):
    info = plsc.get_sparse_core_info()
    ncores, nsub = info.num_cores, info.num_subcores
    nw = ncores * nsub
    bpw = idx.shape[0] // nw
    dim = table.shape[1]
    mesh = plsc.VectorSubcoreMesh(core_axis_name="c", subcore_axis_name="s")

    @functools.partial(
        pl.kernel, mesh=mesh,
        out_type=jax.ShapeDtypeStruct((idx.shape[0], dim), jnp.float32),
        scratch_types=[
            pltpu.VMEM((bpw,), jnp.int32),
            pltpu.VMEM((bpw, dim), jnp.float32),
            pltpu.SemaphoreType.DMA,
        ],
    )
    def k(table_hbm, idx_hbm, out_hbm, idx_v, rows_v, sem):
        wid = lax.axis_index("s") * ncores + lax.axis_index("c")
        base = wid * bpw
        pltpu.sync_copy(idx_hbm.at[pl.ds(base, bpw)], idx_v)
        for j in range(bpw // _GATHER_CHUNK):
            off = j * _GATHER_CHUNK
            pltpu.async_copy(
                table_hbm.at[idx_v.at[pl.ds(off, _GATHER_CHUNK)]],
                rows_v.at[pl.ds(off, _GATHER_CHUNK)], sem).wait()
        pltpu.sync_copy(rows_v, out_hbm.at[pl.ds(base, bpw)])

    return k(table, idx)


def kernel(features, class_codebook, feature_codebook):
    distances, indices, loss11, cperp11, fperp11 = _tc_call(
        features, class_codebook, feature_codebook)

    combined = jnp.concatenate([class_codebook, feature_codebook], axis=0)
    combined = jnp.pad(combined, ((0, 0), (0, 128 - _DIM)))
    nidx = _T * _B * _TOP_K
    npad = 12544
    flat_idx = jnp.concatenate(
        [indices.reshape(-1), jnp.zeros((npad - nidx,), jnp.int32)])
    rows = _sc_gather(combined, flat_idx)
    quantized = rows[:nidx, :_DIM].reshape(_T, _B, _TOP_K, _DIM)

    return (loss11[0, 0], quantized, cperp11[0, 0], fperp11[0, 0],
            indices, distances)


# speedup vs baseline: 1.0338x; 1.0260x over previous
import functools

import jax
import jax.numpy as jnp
from jax import lax
from jax.experimental import pallas as pl
from jax.experimental.pallas import tpu as pltpu
from jax.experimental.pallas import tpu_sc as plsc

_NUM_CLASS = 1024
_NUM_FEAT = 8192
_DIM = 64
_TOP_K = 3
_COMMIT = 0.25
_T = 257
_B = 16
_KTOT = _NUM_CLASS + _NUM_FEAT
_FMAX = float(jnp.finfo(jnp.float32).max)
_TBLK = 8
_ROWS = _TBLK * _B
_NSTEP = (_T + _TBLK - 1) // _TBLK


def _top3(d, width):
    iot = lax.broadcasted_iota(jnp.int32, d.shape, 1)
    cur = d
    vals, idxs = [], []
    onehot = jnp.zeros(d.shape, jnp.float32)
    for _ in range(_TOP_K):
        m = jnp.min(cur, axis=1, keepdims=True)
        ii = jnp.min(jnp.where(cur == m, iot, width), axis=1, keepdims=True)
        oh = iot == ii
        onehot = onehot + oh.astype(jnp.float32)
        vals.append(m)
        idxs.append(ii)
        cur = jnp.where(oh, _FMAX, cur)
    return vals, idxs, onehot


def _pack3(idxs, rows):
    lane = lax.broadcasted_iota(jnp.int32, (rows, _TOP_K), 1)
    return jnp.where(lane == 0, idxs[0],
                     jnp.where(lane == 1, idxs[1], idxs[2]))


def _tc_body(feat_ref, ccb_ref, fcb_ref,
             dist_ref, idx_ref, loss_ref, cperp_ref, fperp_ref,
             c2f, cnt_c, cnt_f, csum, fsum):
    i = pl.program_id(0)

    @pl.when(i == 0)
    def _init():
        fcb = fcb_ref[...]
        c2f[...] = jnp.sum(fcb * fcb, axis=1)[None, :]
        cnt_f[...] = jnp.zeros((_ROWS, _NUM_FEAT), jnp.float32)
        fsum[...] = jnp.zeros((1, 1), jnp.float32)

    x = feat_ref[...].reshape(_ROWS, _DIM)
    m = lax.dot_general(x, fcb_ref[...], (((1,), (1,)), ((), ())),
                        preferred_element_type=jnp.float32)
    x2 = jnp.sum(x * x, axis=1, keepdims=True)
    d = (x2 + c2f[...]) - 2.0 * m

    vals, idxs, onehot = _top3(d, _NUM_FEAT)
    r = lax.broadcasted_iota(jnp.int32, (_ROWS, 1), 0)
    g = _TBLK * i + (r >> 4)
    valid = (g >= 1) & (g <= _T - 1)
    cnt_f[...] += onehot * valid.astype(jnp.float32)
    fsum[...] += jnp.sum(jnp.where(valid, vals[0] + vals[1] + vals[2], 0.0))

    dist_ref[:, :, :_NUM_CLASS] = jnp.full((_TBLK, _B, _NUM_CLASS), _FMAX,
                                           jnp.float32)
    dist_ref[:, :, _NUM_CLASS:] = d.reshape(_TBLK, _B, _NUM_FEAT)
    idx_ref[...] = (_pack3(idxs, _ROWS) + _NUM_CLASS).reshape(
        _TBLK, _B, _TOP_K)

    @pl.when(i == 0)
    def _class_part():
        xc = feat_ref[0]
        mc = lax.dot_general(xc, ccb_ref[...], (((1,), (1,)), ((), ())),
                             preferred_element_type=jnp.float32)
        xc2 = jnp.sum(xc * xc, axis=1, keepdims=True)
        ccb = ccb_ref[...]
        c2c = jnp.sum(ccb * ccb, axis=1)
        dc = (xc2 + c2c[None, :]) - 2.0 * mc
        vals_c, idxs_c, onehot_c = _top3(dc, _NUM_CLASS)
        dist_ref[0, :, :_NUM_CLASS] = dc
        dist_ref[0, :, _NUM_CLASS:] = jnp.full((_B, _NUM_FEAT), _FMAX,
                                               jnp.float32)
        idx_ref[0] = _pack3(idxs_c, _B)
        cnt_c[...] = onehot_c
        csum[...] = jnp.broadcast_to(
            jnp.sum(vals_c[0] + vals_c[1] + vals_c[2]), (1, 1))

    @pl.when(i == _NSTEP - 1)
    def _finalize():
        n_c = float(_B * _TOP_K * _DIM)
        n_f = float((_T - 1) * _B * _TOP_K * _DIM)
        loss_ref[...] = _COMMIT * (csum[...] / n_c + fsum[...] / n_f)
        avg_c = jnp.sum(cnt_c[...], axis=0, keepdims=True) / float(_B)
        s_c = jnp.sum(avg_c * jnp.log(avg_c + 1e-10))
        cperp_ref[...] = jnp.exp(jnp.broadcast_to(-s_c, (1, 1)))
        avg_f = jnp.sum(cnt_f[...], axis=0, keepdims=True) / float(
            (_T - 1) * _B)
        s_f = jnp.sum(avg_f * jnp.log(avg_f + 1e-10))
        fperp_ref[...] = jnp.exp(jnp.broadcast_to(-s_f, (1, 1)))


def _tc_call(features, class_codebook, feature_codebook):
    return pl.pallas_call(
        _tc_body,
        grid=(_NSTEP,),
        in_specs=[
            pl.BlockSpec((_TBLK, _B, _DIM), lambda i: (i, 0, 0)),
            pl.BlockSpec((_NUM_CLASS, _DIM), lambda i: (0, 0)),
            pl.BlockSpec((_NUM_FEAT, _DIM), lambda i: (0, 0)),
        ],
        out_specs=[
            pl.BlockSpec((_TBLK, _B, _KTOT), lambda i: (i, 0, 0)),
            pl.BlockSpec((_TBLK, _B, _TOP_K), lambda i: (i, 0, 0)),
            pl.BlockSpec((1, 1), lambda i: (0, 0)),
            pl.BlockSpec((1, 1), lambda i: (0, 0)),
            pl.BlockSpec((1, 1), lambda i: (0, 0)),
        ],
        out_shape=[
            jax.ShapeDtypeStruct((_T, _B, _KTOT), jnp.float32),
            jax.ShapeDtypeStruct((_T, _B, _TOP_K), jnp.int32),
            jax.ShapeDtypeStruct((1, 1), jnp.float32),
            jax.ShapeDtypeStruct((1, 1), jnp.float32),
            jax.ShapeDtypeStruct((1, 1), jnp.float32),
        ],
        scratch_shapes=[
            pltpu.VMEM((1, _NUM_FEAT), jnp.float32),
            pltpu.VMEM((_B, _NUM_CLASS), jnp.float32),
            pltpu.VMEM((_ROWS, _NUM_FEAT), jnp.float32),
            pltpu.VMEM((1, 1), jnp.float32),
            pltpu.VMEM((1, 1), jnp.float32),
        ],
        compiler_params=pltpu.CompilerParams(
            dimension_semantics=("arbitrary",)),
    )(features, class_codebook, feature_codebook)


_GATHER_CHUNK = 56


def _sc_gather(table, idx):
    info = plsc.get_sparse_core_info()
    ncores, nsub = info.num_cores, info.num_subcores
    nw = ncores * nsub
    bpw = idx.shape[0] // nw
    dim = table.shape[1]
    mesh = plsc.VectorSubcoreMesh(core_axis_name="c", subcore_axis_name="s")

    @functools.partial(
        pl.kernel, mesh=mesh,
        out_type=jax.ShapeDtypeStruct((idx.shape[0], dim), jnp.float32),
        scratch_types=[
            pltpu.VMEM((bpw,), jnp.int32),
            pltpu.VMEM((bpw, dim), jnp.float32),
            pltpu.SemaphoreType.DMA,
        ],
    )
    def k(table_hbm, idx_hbm, out_hbm, idx_v, rows_v, sem):
        wid = lax.axis_index("s") * ncores + lax.axis_index("c")
        base = wid * bpw
        pltpu.sync_copy(idx_hbm.at[pl.ds(base, bpw)], idx_v)
        for j in range(bpw // _GATHER_CHUNK):
            off = j * _GATHER_CHUNK
            pltpu.async_copy(
                table_hbm.at[idx_v.at[pl.ds(off, _GATHER_CHUNK)]],
                rows_v.at[pl.ds(off, _GATHER_CHUNK)], sem).wait()
        pltpu.sync_copy(rows_v, out_hbm.at[pl.ds(base, bpw)])

    return k(table, idx)


def kernel(features, class_codebook, feature_codebook):
    distances, indices, loss11, cperp11, fperp11 = _tc_call(
        features, class_codebook, feature_codebook)

    combined = jnp.concatenate([class_codebook, feature_codebook], axis=0)
    combined = jnp.pad(combined, ((0, 0), (0, 128 - _DIM)))
    nidx = _T * _B * _TOP_K
    npad = 12544
    flat_idx = jnp.concatenate(
        [indices.reshape(-1), jnp.zeros((npad - nidx,), jnp.int32)])
    rows = _sc_gather(combined, flat_idx)
    quantized = rows[:nidx, :_DIM].reshape(_T, _B, _TOP_K, _DIM)

    return (loss11[0, 0], quantized, cperp11[0, 0], fperp11[0, 0],
            indices, distances)


# speedup vs baseline: 1.1469x; 1.1094x over previous
import functools

import jax
import jax.numpy as jnp
from jax import lax
from jax.experimental import pallas as pl
from jax.experimental.pallas import tpu as pltpu
from jax.experimental.pallas import tpu_sc as plsc

_NUM_CLASS = 1024
_NUM_FEAT = 8192
_DIM = 64
_TOP_K = 3
_COMMIT = 0.25
_T = 257
_B = 16
_KTOT = _NUM_CLASS + _NUM_FEAT
_FMAX = float(jnp.finfo(jnp.float32).max)
_TBLK = 8
_ROWS = _TBLK * _B
_NSTEP = (_T + _TBLK - 1) // _TBLK


def _top3(d, width):
    iot = lax.broadcasted_iota(jnp.int32, d.shape, 1)
    cur = d
    vals, idxs = [], []
    for _ in range(_TOP_K):
        m = jnp.min(cur, axis=1, keepdims=True)
        ii = jnp.min(jnp.where(cur == m, iot, width), axis=1, keepdims=True)
        vals.append(m)
        idxs.append(ii)
        cur = jnp.where(iot == ii, _FMAX, cur)
    onehot = (d <= vals[2]).astype(jnp.float32)
    return vals, idxs, onehot


def _pack3(idxs, rows):
    lane = lax.broadcasted_iota(jnp.int32, (rows, _TOP_K), 1)
    return jnp.where(lane == 0, idxs[0],
                     jnp.where(lane == 1, idxs[1], idxs[2]))


def _tc_body(feat_ref, ccb_ref, fcb_ref,
             dist_ref, idx_ref, loss_ref, cperp_ref, fperp_ref,
             c2f, cnt_c, cnt_f, csum, fsum):
    i = pl.program_id(0)

    @pl.when(i == 0)
    def _init():
        fcb = fcb_ref[...]
        c2f[...] = jnp.sum(fcb * fcb, axis=1)[None, :]
        cnt_f[...] = jnp.zeros((_ROWS, _NUM_FEAT), jnp.float32)
        fsum[...] = jnp.zeros((1, 1), jnp.float32)

    x = feat_ref[...]
    m = lax.dot_general(x, fcb_ref[...], (((1,), (1,)), ((), ())),
                        preferred_element_type=jnp.float32)
    x2 = jnp.sum(x * x, axis=1, keepdims=True)
    d = (x2 + c2f[...]) - 2.0 * m

    vals, idxs, onehot = _top3(d, _NUM_FEAT)
    r = lax.broadcasted_iota(jnp.int32, (_ROWS, 1), 0)
    tk = _ROWS * i + r
    valid = (tk >= _B) & (tk < _T * _B)
    cnt_f[...] += onehot * valid.astype(jnp.float32)
    fsum[...] += jnp.sum(jnp.where(valid, vals[0] + vals[1] + vals[2], 0.0))

    dist_ref[:, :_NUM_CLASS] = jnp.full((_ROWS, _NUM_CLASS), _FMAX,
                                        jnp.float32)
    dist_ref[:, _NUM_CLASS:] = d
    idx_ref[...] = _pack3(idxs, _ROWS) + _NUM_CLASS

    @pl.when(i == 0)
    def _class_part():
        xc = feat_ref[:_B, :]
        mc = lax.dot_general(xc, ccb_ref[...], (((1,), (1,)), ((), ())),
                             preferred_element_type=jnp.float32)
        xc2 = jnp.sum(xc * xc, axis=1, keepdims=True)
        ccb = ccb_ref[...]
        c2c = jnp.sum(ccb * ccb, axis=1)
        dc = (xc2 + c2c[None, :]) - 2.0 * mc
        vals_c, idxs_c, onehot_c = _top3(dc, _NUM_CLASS)
        dist_ref[:_B, :_NUM_CLASS] = dc
        dist_ref[:_B, _NUM_CLASS:] = jnp.full((_B, _NUM_FEAT), _FMAX,
                                              jnp.float32)
        idx_ref[:_B, :] = _pack3(idxs_c, _B)
        cnt_c[...] = onehot_c
        csum[...] = jnp.broadcast_to(
            jnp.sum(vals_c[0] + vals_c[1] + vals_c[2]), (1, 1))

    @pl.when(i == _NSTEP - 1)
    def _finalize():
        n_c = float(_B * _TOP_K * _DIM)
        n_f = float((_T - 1) * _B * _TOP_K * _DIM)
        loss_ref[...] = _COMMIT * (csum[...] / n_c + fsum[...] / n_f)
        avg_c = jnp.sum(cnt_c[...], axis=0, keepdims=True) / float(_B)
        s_c = jnp.sum(avg_c * jnp.log(avg_c + 1e-10))
        cperp_ref[...] = jnp.exp(jnp.broadcast_to(-s_c, (1, 1)))
        avg_f = jnp.sum(cnt_f[...], axis=0, keepdims=True) / float(
            (_T - 1) * _B)
        s_f = jnp.sum(avg_f * jnp.log(avg_f + 1e-10))
        fperp_ref[...] = jnp.exp(jnp.broadcast_to(-s_f, (1, 1)))


def _tc_call(features, class_codebook, feature_codebook):
    return pl.pallas_call(
        _tc_body,
        grid=(_NSTEP,),
        in_specs=[
            pl.BlockSpec((_ROWS, _DIM), lambda i: (i, 0)),
            pl.BlockSpec((_NUM_CLASS, _DIM), lambda i: (0, 0)),
            pl.BlockSpec((_NUM_FEAT, _DIM), lambda i: (0, 0)),
        ],
        out_specs=[
            pl.BlockSpec((_ROWS, _KTOT), lambda i: (i, 0)),
            pl.BlockSpec((_ROWS, _TOP_K), lambda i: (i, 0)),
            pl.BlockSpec((1, 1), lambda i: (0, 0)),
            pl.BlockSpec((1, 1), lambda i: (0, 0)),
            pl.BlockSpec((1, 1), lambda i: (0, 0)),
        ],
        out_shape=[
            jax.ShapeDtypeStruct((_T * _B, _KTOT), jnp.float32),
            jax.ShapeDtypeStruct((_T * _B, _TOP_K), jnp.int32),
            jax.ShapeDtypeStruct((1, 1), jnp.float32),
            jax.ShapeDtypeStruct((1, 1), jnp.float32),
            jax.ShapeDtypeStruct((1, 1), jnp.float32),
        ],
        scratch_shapes=[
            pltpu.VMEM((1, _NUM_FEAT), jnp.float32),
            pltpu.VMEM((_B, _NUM_CLASS), jnp.float32),
            pltpu.VMEM((_ROWS, _NUM_FEAT), jnp.float32),
            pltpu.VMEM((1, 1), jnp.float32),
            pltpu.VMEM((1, 1), jnp.float32),
        ],
        compiler_params=pltpu.CompilerParams(
            dimension_semantics=("arbitrary",)),
    )(features, class_codebook, feature_codebook)


_GATHER_CHUNK = 56


def _sc_gather(table, idx):
    info = plsc.get_sparse_core_info()
    ncores, nsub = info.num_cores, info.num_subcores
    nw = ncores * nsub
    bpw = idx.shape[0] // nw
    dim = table.shape[1]
    mesh = plsc.VectorSubcoreMesh(core_axis_name="c", subcore_axis_name="s")

    @functools.partial(
        pl.kernel, mesh=mesh,
        out_type=jax.ShapeDtypeStruct((idx.shape[0], dim), jnp.float32),
        scratch_types=[
            pltpu.VMEM((bpw,), jnp.int32),
            pltpu.VMEM((bpw, dim), jnp.float32),
            pltpu.SemaphoreType.DMA,
        ],
    )
    def k(table_hbm, idx_hbm, out_hbm, idx_v, rows_v, sem):
        wid = lax.axis_index("s") * ncores + lax.axis_index("c")
        base = wid * bpw
        pltpu.sync_copy(idx_hbm.at[pl.ds(base, bpw)], idx_v)
        for j in range(bpw // _GATHER_CHUNK):
            off = j * _GATHER_CHUNK
            pltpu.async_copy(
                table_hbm.at[idx_v.at[pl.ds(off, _GATHER_CHUNK)]],
                rows_v.at[pl.ds(off, _GATHER_CHUNK)], sem).wait()
        pltpu.sync_copy(rows_v, out_hbm.at[pl.ds(base, bpw)])

    return k(table, idx)


def kernel(features, class_codebook, feature_codebook):
    dist2, idx2, loss11, cperp11, fperp11 = _tc_call(
        features.reshape(_T * _B, _DIM), class_codebook, feature_codebook)
    distances = dist2.reshape(_T, _B, _KTOT)
    indices = idx2.reshape(_T, _B, _TOP_K)

    combined = jnp.concatenate([class_codebook, feature_codebook], axis=0)
    combined = jnp.pad(combined, ((0, 0), (0, 128 - _DIM)))
    nidx = _T * _B * _TOP_K
    npad = 12544
    flat_idx = jnp.concatenate(
        [indices.reshape(-1), jnp.zeros((npad - nidx,), jnp.int32)])
    rows = _sc_gather(combined, flat_idx)
    quantized = rows[:nidx, :_DIM].reshape(_T, _B, _TOP_K, _DIM)

    return (loss11[0, 0], quantized, cperp11[0, 0], fperp11[0, 0],
            indices, distances)


# speedup vs baseline: 1.1768x; 1.0261x over previous
import functools

import jax
import jax.numpy as jnp
from jax import lax
from jax.experimental import pallas as pl
from jax.experimental.pallas import tpu as pltpu
from jax.experimental.pallas import tpu_sc as plsc

_NUM_CLASS = 1024
_NUM_FEAT = 8192
_DIM = 64
_TOP_K = 3
_COMMIT = 0.25
_T = 257
_B = 16
_KTOT = _NUM_CLASS + _NUM_FEAT
_FMAX = float(jnp.finfo(jnp.float32).max)
_TBLK = 8
_ROWS = _TBLK * _B
_NSTEP = (_T + _TBLK - 1) // _TBLK


def _top3(d, width):
    iot = lax.broadcasted_iota(jnp.int32, d.shape, 1)
    cur = d
    vals, idxs = [], []
    for _ in range(_TOP_K):
        m = jnp.min(cur, axis=1, keepdims=True)
        ii = jnp.argmin(cur, axis=1, keepdims=True).astype(jnp.int32)
        vals.append(m)
        idxs.append(ii)
        cur = jnp.where(iot == ii, _FMAX, cur)
    onehot = (d <= vals[2]).astype(jnp.float32)
    return vals, idxs, onehot


def _pack3(idxs, rows):
    lane = lax.broadcasted_iota(jnp.int32, (rows, _TOP_K), 1)
    return jnp.where(lane == 0, idxs[0],
                     jnp.where(lane == 1, idxs[1], idxs[2]))


def _tc_body(feat_ref, ccb_ref, fcb_ref,
             dist_ref, idx_ref, loss_ref, cperp_ref, fperp_ref,
             c2f, cnt_c, cnt_f, csum, fsum):
    i = pl.program_id(0)

    @pl.when(i == 0)
    def _init():
        fcb = fcb_ref[...]
        c2f[...] = jnp.sum(fcb * fcb, axis=1)[None, :]
        cnt_f[...] = jnp.zeros((_ROWS, _NUM_FEAT), jnp.float32)
        fsum[...] = jnp.zeros((1, 1), jnp.float32)

    x = feat_ref[...]
    m = lax.dot_general(x, fcb_ref[...], (((1,), (1,)), ((), ())),
                        preferred_element_type=jnp.float32)
    x2 = jnp.sum(x * x, axis=1, keepdims=True)
    d = (x2 + c2f[...]) - 2.0 * m

    vals, idxs, onehot = _top3(d, _NUM_FEAT)
    r = lax.broadcasted_iota(jnp.int32, (_ROWS, 1), 0)
    tk = _ROWS * i + r
    valid = (tk >= _B) & (tk < _T * _B)
    cnt_f[...] += onehot * valid.astype(jnp.float32)
    fsum[...] += jnp.sum(jnp.where(valid, vals[0] + vals[1] + vals[2], 0.0))

    dist_ref[:, :_NUM_CLASS] = jnp.full((_ROWS, _NUM_CLASS), _FMAX,
                                        jnp.float32)
    dist_ref[:, _NUM_CLASS:] = d
    idx_ref[...] = _pack3(idxs, _ROWS) + _NUM_CLASS

    @pl.when(i == 0)
    def _class_part():
        xc = feat_ref[:_B, :]
        mc = lax.dot_general(xc, ccb_ref[...], (((1,), (1,)), ((), ())),
                             preferred_element_type=jnp.float32)
        xc2 = jnp.sum(xc * xc, axis=1, keepdims=True)
        ccb = ccb_ref[...]
        c2c = jnp.sum(ccb * ccb, axis=1)
        dc = (xc2 + c2c[None, :]) - 2.0 * mc
        vals_c, idxs_c, onehot_c = _top3(dc, _NUM_CLASS)
        dist_ref[:_B, :_NUM_CLASS] = dc
        dist_ref[:_B, _NUM_CLASS:] = jnp.full((_B, _NUM_FEAT), _FMAX,
                                              jnp.float32)
        idx_ref[:_B, :] = _pack3(idxs_c, _B)
        cnt_c[...] = onehot_c
        csum[...] = jnp.broadcast_to(
            jnp.sum(vals_c[0] + vals_c[1] + vals_c[2]), (1, 1))

    @pl.when(i == _NSTEP - 1)
    def _finalize():
        n_c = float(_B * _TOP_K * _DIM)
        n_f = float((_T - 1) * _B * _TOP_K * _DIM)
        loss_ref[...] = _COMMIT * (csum[...] / n_c + fsum[...] / n_f)
        avg_c = jnp.sum(cnt_c[...], axis=0, keepdims=True) / float(_B)
        s_c = jnp.sum(avg_c * jnp.log(avg_c + 1e-10))
        cperp_ref[...] = jnp.exp(jnp.broadcast_to(-s_c, (1, 1)))
        avg_f = jnp.sum(cnt_f[...], axis=0, keepdims=True) / float(
            (_T - 1) * _B)
        s_f = jnp.sum(avg_f * jnp.log(avg_f + 1e-10))
        fperp_ref[...] = jnp.exp(jnp.broadcast_to(-s_f, (1, 1)))


def _tc_call(features, class_codebook, feature_codebook):
    return pl.pallas_call(
        _tc_body,
        grid=(_NSTEP,),
        in_specs=[
            pl.BlockSpec((_ROWS, _DIM), lambda i: (i, 0)),
            pl.BlockSpec((_NUM_CLASS, _DIM), lambda i: (0, 0)),
            pl.BlockSpec((_NUM_FEAT, _DIM), lambda i: (0, 0)),
        ],
        out_specs=[
            pl.BlockSpec((_ROWS, _KTOT), lambda i: (i, 0)),
            pl.BlockSpec((_ROWS, _TOP_K), lambda i: (i, 0)),
            pl.BlockSpec((1, 1), lambda i: (0, 0)),
            pl.BlockSpec((1, 1), lambda i: (0, 0)),
            pl.BlockSpec((1, 1), lambda i: (0, 0)),
        ],
        out_shape=[
            jax.ShapeDtypeStruct((_T * _B, _KTOT), jnp.float32),
            jax.ShapeDtypeStruct((_T * _B, _TOP_K), jnp.int32),
            jax.ShapeDtypeStruct((1, 1), jnp.float32),
            jax.ShapeDtypeStruct((1, 1), jnp.float32),
            jax.ShapeDtypeStruct((1, 1), jnp.float32),
        ],
        scratch_shapes=[
            pltpu.VMEM((1, _NUM_FEAT), jnp.float32),
            pltpu.VMEM((_B, _NUM_CLASS), jnp.float32),
            pltpu.VMEM((_ROWS, _NUM_FEAT), jnp.float32),
            pltpu.VMEM((1, 1), jnp.float32),
            pltpu.VMEM((1, 1), jnp.float32),
        ],
        compiler_params=pltpu.CompilerParams(
            dimension_semantics=("arbitrary",)),
    )(features, class_codebook, feature_codebook)


_GATHER_CHUNK = 56


def _sc_gather(table, idx):
    info = plsc.get_sparse_core_info()
    ncores, nsub = info.num_cores, info.num_subcores
    nw = ncores * nsub
    bpw = idx.shape[0] // nw
    dim = table.shape[1]
    mesh = plsc.VectorSubcoreMesh(core_axis_name="c", subcore_axis_name="s")

    @functools.partial(
        pl.kernel, mesh=mesh,
        out_type=jax.ShapeDtypeStruct((idx.shape[0], dim), jnp.float32),
        scratch_types=[
            pltpu.VMEM((bpw,), jnp.int32),
            pltpu.VMEM((bpw, dim), jnp.float32),
            pltpu.SemaphoreType.DMA,
        ],
    )
    def k(table_hbm, idx_hbm, out_hbm, idx_v, rows_v, sem):
        wid = lax.axis_index("s") * ncores + lax.axis_index("c")
        base = wid * bpw
        pltpu.sync_copy(idx_hbm.at[pl.ds(base, bpw)], idx_v)
        for j in range(bpw // _GATHER_CHUNK):
            off = j * _GATHER_CHUNK
            pltpu.async_copy(
                table_hbm.at[idx_v.at[pl.ds(off, _GATHER_CHUNK)]],
                rows_v.at[pl.ds(off, _GATHER_CHUNK)], sem).wait()
        pltpu.sync_copy(rows_v, out_hbm.at[pl.ds(base, bpw)])

    return k(table, idx)


def kernel(features, class_codebook, feature_codebook):
    dist2, idx2, loss11, cperp11, fperp11 = _tc_call(
        features.reshape(_T * _B, _DIM), class_codebook, feature_codebook)
    distances = dist2.reshape(_T, _B, _KTOT)
    indices = idx2.reshape(_T, _B, _TOP_K)

    combined = jnp.concatenate([class_codebook, feature_codebook], axis=0)
    combined = jnp.pad(combined, ((0, 0), (0, 128 - _DIM)))
    nidx = _T * _B * _TOP_K
    npad = 12544
    flat_idx = jnp.concatenate(
        [indices.reshape(-1), jnp.zeros((npad - nidx,), jnp.int32)])
    rows = _sc_gather(combined, flat_idx)
    quantized = rows[:nidx, :_DIM].reshape(_T, _B, _TOP_K, _DIM)

    return (loss11[0, 0], quantized, cperp11[0, 0], fperp11[0, 0],
            indices, distances)


# speedup vs baseline: 1.2108x; 1.0289x over previous
import functools

import jax
import jax.numpy as jnp
from jax import lax
from jax.experimental import pallas as pl
from jax.experimental.pallas import tpu as pltpu
from jax.experimental.pallas import tpu_sc as plsc

_NUM_CLASS = 1024
_NUM_FEAT = 8192
_DIM = 64
_TOP_K = 3
_COMMIT = 0.25
_T = 257
_B = 16
_KTOT = _NUM_CLASS + _NUM_FEAT
_FMAX = float(jnp.finfo(jnp.float32).max)
_TBLK = 8
_ROWS = _TBLK * _B
_NSTEP = (_T + _TBLK - 1) // _TBLK


def _top3(d, width):
    iot = lax.broadcasted_iota(jnp.int32, d.shape, 1)
    cur = d
    vals, idxs = [], []
    for _ in range(_TOP_K):
        m = jnp.min(cur, axis=1, keepdims=True)
        ii = jnp.argmin(cur, axis=1, keepdims=True).astype(jnp.int32)
        vals.append(m)
        idxs.append(ii)
        cur = jnp.where(iot == ii, _FMAX, cur)
    onehot = (d <= vals[2]).astype(jnp.float32)
    return vals, idxs, onehot


def _pack3(idxs, rows):
    lane = lax.broadcasted_iota(jnp.int32, (rows, _TOP_K), 1)
    return jnp.where(lane == 0, idxs[0],
                     jnp.where(lane == 1, idxs[1], idxs[2]))


def _tc_body(feat_ref, ccb_ref, fcb_ref,
             dist_ref, idx_ref, loss_ref, cperp_ref, fperp_ref,
             c2f, cnt_c, cnt_f, csum, fsum):
    i = pl.program_id(0)

    @pl.when(i == 0)
    def _init():
        fcb = fcb_ref[...]
        c2f[...] = jnp.sum(fcb * fcb, axis=1)[None, :]
        cnt_f[...] = jnp.zeros((1, _NUM_FEAT), jnp.float32)
        fsum[...] = jnp.zeros((1, 1), jnp.float32)

    x = feat_ref[...]
    m = lax.dot_general(x, fcb_ref[...], (((1,), (1,)), ((), ())),
                        preferred_element_type=jnp.float32)
    x2 = jnp.sum(x * x, axis=1, keepdims=True)
    d = (x2 + c2f[...]) - 2.0 * m

    vals, idxs, onehot = _top3(d, _NUM_FEAT)
    r = lax.broadcasted_iota(jnp.int32, (_ROWS, 1), 0)
    tk = _ROWS * i + r
    valid = (tk >= _B) & (tk < _T * _B)
    cnt_f[...] += jnp.sum(onehot * valid.astype(jnp.float32), axis=0,
                          keepdims=True)
    fsum[...] += jnp.sum(jnp.where(valid, vals[0] + vals[1] + vals[2], 0.0))

    dist_ref[:, :_NUM_CLASS] = jnp.full((_ROWS, _NUM_CLASS), _FMAX,
                                        jnp.float32)
    dist_ref[:, _NUM_CLASS:] = d
    idx_ref[...] = _pack3(idxs, _ROWS) + _NUM_CLASS

    @pl.when(i == 0)
    def _class_part():
        xc = feat_ref[:_B, :]
        mc = lax.dot_general(xc, ccb_ref[...], (((1,), (1,)), ((), ())),
                             preferred_element_type=jnp.float32)
        xc2 = jnp.sum(xc * xc, axis=1, keepdims=True)
        ccb = ccb_ref[...]
        c2c = jnp.sum(ccb * ccb, axis=1)
        dc = (xc2 + c2c[None, :]) - 2.0 * mc
        vals_c, idxs_c, onehot_c = _top3(dc, _NUM_CLASS)
        dist_ref[:_B, :_NUM_CLASS] = dc
        dist_ref[:_B, _NUM_CLASS:] = jnp.full((_B, _NUM_FEAT), _FMAX,
                                              jnp.float32)
        idx_ref[:_B, :] = _pack3(idxs_c, _B)
        cnt_c[...] = onehot_c
        csum[...] = jnp.broadcast_to(
            jnp.sum(vals_c[0] + vals_c[1] + vals_c[2]), (1, 1))

    @pl.when(i == _NSTEP - 1)
    def _finalize():
        n_c = float(_B * _TOP_K * _DIM)
        n_f = float((_T - 1) * _B * _TOP_K * _DIM)
        loss_ref[...] = _COMMIT * (csum[...] / n_c + fsum[...] / n_f)
        avg_c = jnp.sum(cnt_c[...], axis=0, keepdims=True) / float(_B)
        s_c = jnp.sum(avg_c * jnp.log(avg_c + 1e-10))
        cperp_ref[...] = jnp.exp(jnp.broadcast_to(-s_c, (1, 1)))
        avg_f = cnt_f[...] / float((_T - 1) * _B)
        s_f = jnp.sum(avg_f * jnp.log(avg_f + 1e-10))
        fperp_ref[...] = jnp.exp(jnp.broadcast_to(-s_f, (1, 1)))


def _tc_call(features, class_codebook, feature_codebook):
    return pl.pallas_call(
        _tc_body,
        grid=(_NSTEP,),
        in_specs=[
            pl.BlockSpec((_ROWS, _DIM), lambda i: (i, 0)),
            pl.BlockSpec((_NUM_CLASS, _DIM), lambda i: (0, 0)),
            pl.BlockSpec((_NUM_FEAT, _DIM), lambda i: (0, 0)),
        ],
        out_specs=[
            pl.BlockSpec((_ROWS, _KTOT), lambda i: (i, 0)),
            pl.BlockSpec((_ROWS, _TOP_K), lambda i: (i, 0)),
            pl.BlockSpec((1, 1), lambda i: (0, 0)),
            pl.BlockSpec((1, 1), lambda i: (0, 0)),
            pl.BlockSpec((1, 1), lambda i: (0, 0)),
        ],
        out_shape=[
            jax.ShapeDtypeStruct((_T * _B, _KTOT), jnp.float32),
            jax.ShapeDtypeStruct((_T * _B, _TOP_K), jnp.int32),
            jax.ShapeDtypeStruct((1, 1), jnp.float32),
            jax.ShapeDtypeStruct((1, 1), jnp.float32),
            jax.ShapeDtypeStruct((1, 1), jnp.float32),
        ],
        scratch_shapes=[
            pltpu.VMEM((1, _NUM_FEAT), jnp.float32),
            pltpu.VMEM((_B, _NUM_CLASS), jnp.float32),
            pltpu.VMEM((1, _NUM_FEAT), jnp.float32),
            pltpu.VMEM((1, 1), jnp.float32),
            pltpu.VMEM((1, 1), jnp.float32),
        ],
        compiler_params=pltpu.CompilerParams(
            dimension_semantics=("arbitrary",)),
    )(features, class_codebook, feature_codebook)


_GATHER_CHUNK = 56


def _sc_gather(table, idx):
    info = plsc.get_sparse_core_info()
    ncores, nsub = info.num_cores, info.num_subcores
    nw = ncores * nsub
    bpw = idx.shape[0] // nw
    dim = table.shape[1]
    mesh = plsc.VectorSubcoreMesh(core_axis_name="c", subcore_axis_name="s")

    @functools.partial(
        pl.kernel, mesh=mesh,
        out_type=jax.ShapeDtypeStruct((idx.shape[0], dim), jnp.float32),
        scratch_types=[
            pltpu.VMEM((bpw,), jnp.int32),
            pltpu.VMEM((bpw, dim), jnp.float32),
            pltpu.SemaphoreType.DMA,
        ],
    )
    def k(table_hbm, idx_hbm, out_hbm, idx_v, rows_v, sem):
        wid = lax.axis_index("s") * ncores + lax.axis_index("c")
        base = wid * bpw
        pltpu.sync_copy(idx_hbm.at[pl.ds(base, bpw)], idx_v)
        for j in range(bpw // _GATHER_CHUNK):
            off = j * _GATHER_CHUNK
            pltpu.async_copy(
                table_hbm.at[idx_v.at[pl.ds(off, _GATHER_CHUNK)]],
                rows_v.at[pl.ds(off, _GATHER_CHUNK)], sem).wait()
        pltpu.sync_copy(rows_v, out_hbm.at[pl.ds(base, bpw)])

    return k(table, idx)


def kernel(features, class_codebook, feature_codebook):
    dist2, idx2, loss11, cperp11, fperp11 = _tc_call(
        features.reshape(_T * _B, _DIM), class_codebook, feature_codebook)
    distances = dist2.reshape(_T, _B, _KTOT)
    indices = idx2.reshape(_T, _B, _TOP_K)

    combined = jnp.concatenate([class_codebook, feature_codebook], axis=0)
    combined = jnp.pad(combined, ((0, 0), (0, 128 - _DIM)))
    nidx = _T * _B * _TOP_K
    npad = 12544
    flat_idx = jnp.concatenate(
        [indices.reshape(-1), jnp.zeros((npad - nidx,), jnp.int32)])
    rows = _sc_gather(combined, flat_idx)
    quantized = rows[:nidx, :_DIM].reshape(_T, _B, _TOP_K, _DIM)

    return (loss11[0, 0], quantized, cperp11[0, 0], fperp11[0, 0],
            indices, distances)
